# baseline (speedup 1.0000x reference)
import functools

import jax
import jax.numpy as jnp
from jax import lax
from jax.experimental import pallas as pl
from jax.experimental.pallas import tpu as pltpu
from jax.experimental.pallas import tpu_sc as plsc

_K = 1024
_D = 64
_N = 16 * 1024
_R = 512
_GRID = _N // _R

_NC = 2
_NS = 16
_NW = _NC * _NS
_BPW = _N // _NW
_CHUNK = 64


def _tc_body(x_ref, w_ref, wsq_ref, xsq_ref, dist_ref, idx_ref, q_ref, counts_ref, loss_ref):
    i = pl.program_id(0)
    xb = x_ref[...]
    w = w_ref[...]
    xsq = xsq_ref[...]
    wsq = wsq_ref[...]
    mm = jax.lax.dot_general(
        xb.astype(jnp.bfloat16), w,
        dimension_numbers=(((1,), (1,)), ((), ())),
        preferred_element_type=jnp.float32)
    dist = xsq + wsq - 2.0 * mm
    dist_ref[...] = dist

    mind = jnp.min(dist, axis=1, keepdims=True)
    kiota = jax.lax.broadcasted_iota(jnp.int32, (_R, _K), 1)
    idx = jnp.min(jnp.where(dist == mind, kiota, _K), axis=1)
    idx_ref[...] = idx[:, None]

    one_hot = (kiota == idx[:, None]).astype(jnp.float32)
    q = jnp.dot(one_hot, w, preferred_element_type=jnp.float32)
    q_ref[...] = q

    @pl.when(i == 0)
    def _init():
        counts_ref[...] = jnp.zeros_like(counts_ref)
        loss_ref[...] = jnp.zeros_like(loss_ref)

    counts_ref[...] += jnp.sum(one_hot, axis=0, keepdims=True)
    s = jnp.sum((q - xb) ** 2)
    lane = jax.lax.broadcasted_iota(jnp.int32, (1, 128), 1)
    loss_ref[...] += jnp.where(lane == 0, s, 0.0)


def _sc_body(idx_hbm, zeros_hbm, enc_hbm, idx_v, oh_v):
    wid = lax.axis_index("s") * _NC + lax.axis_index("c")
    base = wid * _BPW
    pltpu.sync_copy(idx_hbm.at[pl.ds(base, _BPW)], idx_v)

    pltpu.sync_copy(zeros_hbm, oh_v)

    ones = jnp.ones((16,), jnp.float32)
    zvec = jnp.zeros((16,), jnp.float32)
    liota = jax.lax.iota(jnp.int32, 16)
    for c in range(_BPW // _CHUNK):
        flats = []
        for g in range(_CHUNK // 16):
            r = c * _CHUNK + g * 16
            cols = idx_v[pl.ds(r, 16)]
            flat = (liota + (g * 16)) * _K + cols
            flats.append(flat)
            plsc.store_scatter(oh_v, [flat], ones)
        pltpu.sync_copy(
            oh_v, enc_hbm.at[pl.ds((base + c * _CHUNK) * _K, _CHUNK * _K)])
        for flat in flats:
            plsc.store_scatter(oh_v, [flat], zvec)


def _sc_encode(idx_flat, zeros):
    mesh = plsc.VectorSubcoreMesh(core_axis_name="c", subcore_axis_name="s",
                                  num_cores=_NC, num_subcores=_NS)
    f = functools.partial(
        pl.kernel, _sc_body, mesh=mesh,
        out_type=jax.ShapeDtypeStruct((_N * _K,), jnp.float32),
        scratch_types=[
            pltpu.VMEM((_BPW,), jnp.int32),
            pltpu.VMEM((_CHUNK * _K,), jnp.float32),
        ],
        compiler_params=pltpu.CompilerParams(needs_layout_passes=False),
    )()
    return f(idx_flat, zeros)


def kernel(inputs, weight):
    x = jnp.transpose(inputs, (0, 2, 1)).reshape(_N, _D)

    dist, idx, q, counts, losspart = pl.pallas_call(
        _tc_body,
        grid=(_GRID,),
        in_specs=[
            pl.BlockSpec((_R, _D), lambda i: (i, 0)),
            pl.BlockSpec((_K, _D), lambda i: (0, 0)),
            pl.BlockSpec((1, _K), lambda i: (0, 0)),
            pl.BlockSpec((_R, 1), lambda i: (i, 0)),
        ],
        out_specs=[
            pl.BlockSpec((_R, _K), lambda i: (i, 0)),
            pl.BlockSpec((_R, 1), lambda i: (i, 0)),
            pl.BlockSpec((_R, _D), lambda i: (i, 0)),
            pl.BlockSpec((1, _K), lambda i: (0, 0)),
            pl.BlockSpec((1, 128), lambda i: (0, 0)),
        ],
        out_shape=[
            jax.ShapeDtypeStruct((_N, _K), jnp.float32),
            jax.ShapeDtypeStruct((_N, 1), jnp.int32),
            jax.ShapeDtypeStruct((_N, _D), jnp.float32),
            jax.ShapeDtypeStruct((1, _K), jnp.float32),
            jax.ShapeDtypeStruct((1, 128), jnp.float32),
        ],
    )(x, weight, jnp.sum(weight ** 2, axis=1)[None, :],
      jnp.sum(x ** 2, axis=1)[:, None])

    zeros = jnp.zeros((_CHUNK * _K,), jnp.float32)
    enc = _sc_encode(idx.reshape(_N), zeros)

    n_elems = jnp.float32(_N * _D)
    e_latent = losspart[0, 0] / n_elems
    vq_loss = e_latent + 0.25 * e_latent

    avg_probs = counts[0] / jnp.float32(_N)
    perplexity = jnp.exp(-jnp.sum(avg_probs * jnp.log(avg_probs + 1e-10)))

    quantized_st = jnp.transpose(q.reshape(16, 1024, _D), (0, 2, 1))
    encodings = enc.reshape(_D, 1024, -1)
    distances = dist.reshape(_D, 1024, -1)
    return (vq_loss, quantized_st, perplexity, encodings, distances, idx)


# speedup vs baseline: 1.0245x; 1.0245x over previous
import functools

import jax
import jax.numpy as jnp
from jax import lax
from jax.experimental import pallas as pl
from jax.experimental.pallas import tpu as pltpu
from jax.experimental.pallas import tpu_sc as plsc

_K = 1024
_D = 64
_N = 16 * 1024
_R = 1024
_GRID = _N // _R

_NC = 2
_NS = 16
_NW = _NC * _NS
_BPW = _N // _NW
_CHUNK = 64


def _tc_body(x_ref, w_ref, wsq_ref, xsq_ref, dist_ref, idx_ref, q_ref, counts_ref, loss_ref):
    i = pl.program_id(0)
    xb = x_ref[...]
    w = w_ref[...]
    xsq = xsq_ref[...]
    wsq = wsq_ref[...]
    mm = jax.lax.dot_general(
        xb.astype(jnp.bfloat16), w,
        dimension_numbers=(((1,), (1,)), ((), ())),
        preferred_element_type=jnp.float32)
    dist = xsq + wsq - 2.0 * mm
    dist_ref[...] = dist

    mind = jnp.min(dist, axis=1, keepdims=True)
    kiota = jax.lax.broadcasted_iota(jnp.int32, (_R, _K), 1)
    idx = jnp.min(jnp.where(dist == mind, kiota, _K), axis=1)
    idx_ref[...] = idx[:, None]

    one_hot = (kiota == idx[:, None]).astype(jnp.float32)
    q = jnp.dot(one_hot, w, preferred_element_type=jnp.float32)
    q_ref[...] = q

    @pl.when(i == 0)
    def _init():
        counts_ref[...] = jnp.zeros_like(counts_ref)
        loss_ref[...] = jnp.zeros_like(loss_ref)

    counts_ref[...] += jnp.sum(one_hot, axis=0, keepdims=True)
    s = jnp.sum((q - xb) ** 2)
    lane = jax.lax.broadcasted_iota(jnp.int32, (1, 128), 1)
    loss_ref[...] += jnp.where(lane == 0, s, 0.0)


def _sc_body(idx_hbm, zeros_hbm, enc_hbm, idx_v, oh_v):
    wid = lax.axis_index("s") * _NC + lax.axis_index("c")
    base = wid * _BPW
    pltpu.sync_copy(idx_hbm.at[pl.ds(base, _BPW)], idx_v)

    pltpu.sync_copy(zeros_hbm, oh_v)

    ones = jnp.ones((16,), jnp.float32)
    zvec = jnp.zeros((16,), jnp.float32)
    liota = jax.lax.iota(jnp.int32, 16)
    for c in range(_BPW // _CHUNK):
        flats = []
        for g in range(_CHUNK // 16):
            r = c * _CHUNK + g * 16
            cols = idx_v[pl.ds(r, 16)]
            flat = (liota + (g * 16)) * _K + cols
            flats.append(flat)
            plsc.store_scatter(oh_v, [flat], ones)
        pltpu.sync_copy(
            oh_v, enc_hbm.at[pl.ds((base + c * _CHUNK) * _K, _CHUNK * _K)])
        for flat in flats:
            plsc.store_scatter(oh_v, [flat], zvec)


def _sc_encode(idx_flat, zeros):
    mesh = plsc.VectorSubcoreMesh(core_axis_name="c", subcore_axis_name="s",
                                  num_cores=_NC, num_subcores=_NS)
    f = functools.partial(
        pl.kernel, _sc_body, mesh=mesh,
        out_type=jax.ShapeDtypeStruct((_N * _K,), jnp.float32),
        scratch_types=[
            pltpu.VMEM((_BPW,), jnp.int32),
            pltpu.VMEM((_CHUNK * _K,), jnp.float32),
        ],
        compiler_params=pltpu.CompilerParams(needs_layout_passes=False),
    )()
    return f(idx_flat, zeros)


def kernel(inputs, weight):
    x = jnp.transpose(inputs, (0, 2, 1)).reshape(_N, _D)

    dist, idx, q, counts, losspart = pl.pallas_call(
        _tc_body,
        grid=(_GRID,),
        in_specs=[
            pl.BlockSpec((_R, _D), lambda i: (i, 0)),
            pl.BlockSpec((_K, _D), lambda i: (0, 0)),
            pl.BlockSpec((1, _K), lambda i: (0, 0)),
            pl.BlockSpec((_R, 1), lambda i: (i, 0)),
        ],
        out_specs=[
            pl.BlockSpec((_R, _K), lambda i: (i, 0)),
            pl.BlockSpec((_R, 1), lambda i: (i, 0)),
            pl.BlockSpec((_R, _D), lambda i: (i, 0)),
            pl.BlockSpec((1, _K), lambda i: (0, 0)),
            pl.BlockSpec((1, 128), lambda i: (0, 0)),
        ],
        out_shape=[
            jax.ShapeDtypeStruct((_N, _K), jnp.float32),
            jax.ShapeDtypeStruct((_N, 1), jnp.int32),
            jax.ShapeDtypeStruct((_N, _D), jnp.float32),
            jax.ShapeDtypeStruct((1, _K), jnp.float32),
            jax.ShapeDtypeStruct((1, 128), jnp.float32),
        ],
    )(x, weight, jnp.sum(weight ** 2, axis=1)[None, :],
      jnp.sum(x ** 2, axis=1)[:, None])

    zeros = jnp.zeros((_CHUNK * _K,), jnp.float32)
    enc = _sc_encode(idx.reshape(_N), zeros)

    n_elems = jnp.float32(_N * _D)
    e_latent = losspart[0, 0] / n_elems
    vq_loss = e_latent + 0.25 * e_latent

    avg_probs = counts[0] / jnp.float32(_N)
    perplexity = jnp.exp(-jnp.sum(avg_probs * jnp.log(avg_probs + 1e-10)))

    quantized_st = jnp.transpose(q.reshape(16, 1024, _D), (0, 2, 1))
    encodings = enc.reshape(_D, 1024, -1)
    distances = dist.reshape(_D, 1024, -1)
    return (vq_loss, quantized_st, perplexity, encodings, distances, idx)


# speedup vs baseline: 1.1035x; 1.0772x over previous
import functools

import jax
import jax.numpy as jnp
from jax import lax
from jax.experimental import pallas as pl
from jax.experimental.pallas import tpu as pltpu
from jax.experimental.pallas import tpu_sc as plsc

_K = 1024
_D = 64
_N = 16 * 1024
_R = 1024
_GRID = _N // _R

_NC = 2
_NS = 16
_NW = _NC * _NS
_BPW = _N // _NW
_CHUNK = 32


def _tc_body(x_ref, w_ref, wsq_ref, xsq_ref, dist_ref, idx_ref, loss_ref):
    i = pl.program_id(0)
    xb = x_ref[...]
    w = w_ref[...]
    xsq = xsq_ref[...]
    wsq = wsq_ref[...]
    mm = jax.lax.dot_general(
        xb.astype(jnp.bfloat16), w,
        dimension_numbers=(((1,), (1,)), ((), ())),
        preferred_element_type=jnp.float32)
    dist = xsq + wsq - 2.0 * mm
    dist_ref[...] = dist

    mind = jnp.min(dist, axis=1, keepdims=True)
    kiota = jax.lax.broadcasted_iota(jnp.int32, (_R, _K), 1)
    idx = jnp.min(jnp.where(dist == mind, kiota, _K), axis=1)
    idx_ref[...] = idx[:, None]

    @pl.when(i == 0)
    def _init():
        loss_ref[...] = jnp.zeros_like(loss_ref)

    s = jnp.sum(mind)
    lane = jax.lax.broadcasted_iota(jnp.int32, (1, 128), 1)
    loss_ref[...] += jnp.where(lane == 0, s, 0.0)


def _sc_body(wpad_hbm, idx_hbm, zeros_hbm, enc_hbm, qpad_hbm, counts_hbm,
             idx_v, rows_v, oh_v, counts_v, sem):
    wid = lax.axis_index("s") * _NC + lax.axis_index("c")
    base = wid * _BPW
    pltpu.sync_copy(idx_hbm.at[pl.ds(base, _BPW)], idx_v)

    gathers = [
        pltpu.async_copy(wpad_hbm.at[idx_v.at[pl.ds(128 * j, 128)]],
                         rows_v.at[pl.ds(128 * j, 128), :], sem)
        for j in range(4)
    ]

    pltpu.sync_copy(zeros_hbm.at[pl.ds(0, _CHUNK * _K)], oh_v)
    pltpu.sync_copy(zeros_hbm.at[pl.ds(0, _K)], counts_v)

    ones = jnp.ones((16,), jnp.float32)
    zvec = jnp.zeros((16,), jnp.float32)
    liota = jax.lax.iota(jnp.int32, 16)
    for c in range(_BPW // _CHUNK):
        flats = []
        for g in range(_CHUNK // 16):
            r = c * _CHUNK + g * 16
            cols = idx_v[pl.ds(r, 16)]
            flat = (liota + (g * 16)) * _K + cols
            flats.append(flat)
            plsc.store_scatter(oh_v, [flat], ones)
            plsc.addupdate_scatter(counts_v, [cols], ones)
        pltpu.sync_copy(
            oh_v, enc_hbm.at[pl.ds((base + c * _CHUNK) * _K, _CHUNK * _K)])
        for flat in flats:
            plsc.store_scatter(oh_v, [flat], zvec)

    pltpu.sync_copy(counts_v, counts_hbm.at[wid])
    for cp in gathers:
        cp.wait()
    pltpu.sync_copy(rows_v, qpad_hbm.at[pl.ds(base, _BPW), :])


def _sc_encode(wpad, idx_flat, zeros):
    mesh = plsc.VectorSubcoreMesh(core_axis_name="c", subcore_axis_name="s",
                                  num_cores=_NC, num_subcores=_NS)
    f = functools.partial(
        pl.kernel, _sc_body, mesh=mesh,
        out_type=[
            jax.ShapeDtypeStruct((_N * _K,), jnp.float32),
            jax.ShapeDtypeStruct((_N, 128), jnp.float32),
            jax.ShapeDtypeStruct((_NW, _K), jnp.float32),
        ],
        scratch_types=[
            pltpu.VMEM((_BPW,), jnp.int32),
            pltpu.VMEM((_BPW, 128), jnp.float32),
            pltpu.VMEM((_CHUNK * _K,), jnp.float32),
            pltpu.VMEM((_K,), jnp.float32),
            pltpu.SemaphoreType.DMA,
        ],
        compiler_params=pltpu.CompilerParams(needs_layout_passes=False),
    )()
    return f(wpad, idx_flat, zeros)


def kernel(inputs, weight):
    x = jnp.transpose(inputs, (0, 2, 1)).reshape(_N, _D)

    dist, idx, losspart = pl.pallas_call(
        _tc_body,
        grid=(_GRID,),
        in_specs=[
            pl.BlockSpec((_R, _D), lambda i: (i, 0)),
            pl.BlockSpec((_K, _D), lambda i: (0, 0)),
            pl.BlockSpec((1, _K), lambda i: (0, 0)),
            pl.BlockSpec((_R, 1), lambda i: (i, 0)),
        ],
        out_specs=[
            pl.BlockSpec((_R, _K), lambda i: (i, 0)),
            pl.BlockSpec((_R, 1), lambda i: (i, 0)),
            pl.BlockSpec((1, 128), lambda i: (0, 0)),
        ],
        out_shape=[
            jax.ShapeDtypeStruct((_N, _K), jnp.float32),
            jax.ShapeDtypeStruct((_N, 1), jnp.int32),
            jax.ShapeDtypeStruct((1, 128), jnp.float32),
        ],
    )(x, weight, jnp.sum(weight ** 2, axis=1)[None, :],
      jnp.sum(x ** 2, axis=1)[:, None])

    zeros = jnp.zeros((_CHUNK * _K,), jnp.float32)
    wpad = jnp.pad(weight, ((0, 0), (0, 128 - _D)))
    enc, qpad, counts32 = _sc_encode(wpad, idx.reshape(_N), zeros)

    n_elems = jnp.float32(_N * _D)
    e_latent = losspart[0, 0] / n_elems
    vq_loss = e_latent + 0.25 * e_latent

    counts = jnp.sum(counts32, axis=0)
    avg_probs = counts / jnp.float32(_N)
    perplexity = jnp.exp(-jnp.sum(avg_probs * jnp.log(avg_probs + 1e-10)))

    q = qpad[:, :_D]
    quantized_st = jnp.transpose(q.reshape(16, 1024, _D), (0, 2, 1))
    encodings = enc.reshape(_D, 1024, -1)
    distances = dist.reshape(_D, 1024, -1)
    return (vq_loss, quantized_st, perplexity, encodings, distances, idx)


# speedup vs baseline: 2.0973x; 1.9005x over previous
import functools

import jax
import jax.numpy as jnp
from jax import lax
from jax.experimental import pallas as pl
from jax.experimental.pallas import tpu as pltpu
from jax.experimental.pallas import tpu_sc as plsc

_K = 1024
_D = 64
_N = 16 * 1024
_R = 1024
_GRID = _N // _R
_SLABS = _R * _K // (1024 * 256)

_NC = 2
_NS = 16
_NW = _NC * _NS
_BPW = _N // _NW


def _tc_body(x_ref, w_ref, wsq_ref, xsq_ref, dist_ref, enc_ref, idx_ref,
             loss_ref):
    i = pl.program_id(0)
    xb = x_ref[...]
    w = w_ref[...]
    xsq = xsq_ref[...]
    wsq = wsq_ref[...]
    mm = jax.lax.dot_general(
        xb.astype(jnp.bfloat16), w,
        dimension_numbers=(((1,), (1,)), ((), ())),
        preferred_element_type=jnp.float32)
    dist = xsq + wsq - 2.0 * mm
    dist_ref[...] = dist.reshape(_SLABS, 1024, 256)

    mind = jnp.min(dist, axis=1, keepdims=True)
    kiota = jax.lax.broadcasted_iota(jnp.int32, (_R, _K), 1)
    idx = jnp.min(jnp.where(dist == mind, kiota, _K), axis=1)
    idx_ref[...] = idx[:, None]

    one_hot = (kiota == idx[:, None]).astype(jnp.float32)
    enc_ref[...] = one_hot.reshape(_SLABS, 1024, 256)

    @pl.when(i == 0)
    def _init():
        loss_ref[...] = jnp.zeros_like(loss_ref)

    s = jnp.sum(mind)
    lane = jax.lax.broadcasted_iota(jnp.int32, (1, 128), 1)
    loss_ref[...] += jnp.where(lane == 0, s, 0.0)


def _sc_body(wpad_hbm, idx_hbm, zeros_hbm, qpad_hbm, counts_hbm,
             idx_v, rows_v, counts_v, sem):
    wid = lax.axis_index("s") * _NC + lax.axis_index("c")
    base = wid * _BPW
    pltpu.sync_copy(idx_hbm.at[pl.ds(base, _BPW)], idx_v)

    gathers = [
        pltpu.async_copy(wpad_hbm.at[idx_v.at[pl.ds(128 * j, 128)]],
                         rows_v.at[pl.ds(128 * j, 128), :], sem)
        for j in range(4)
    ]

    pltpu.sync_copy(zeros_hbm, counts_v)
    ones = jnp.ones((16,), jnp.float32)
    for g in range(_BPW // 16):
        cols = idx_v[pl.ds(g * 16, 16)]
        plsc.addupdate_scatter(counts_v, [cols], ones)
    pltpu.sync_copy(counts_v, counts_hbm.at[wid])

    for cp in gathers:
        cp.wait()
    pltpu.sync_copy(rows_v, qpad_hbm.at[pl.ds(base, _BPW), :])


def _sc_encode(wpad, idx_flat, zeros):
    mesh = plsc.VectorSubcoreMesh(core_axis_name="c", subcore_axis_name="s",
                                  num_cores=_NC, num_subcores=_NS)
    f = functools.partial(
        pl.kernel, _sc_body, mesh=mesh,
        out_type=[
            jax.ShapeDtypeStruct((_N, 128), jnp.float32),
            jax.ShapeDtypeStruct((_NW, _K), jnp.float32),
        ],
        scratch_types=[
            pltpu.VMEM((_BPW,), jnp.int32),
            pltpu.VMEM((_BPW, 128), jnp.float32),
            pltpu.VMEM((_K,), jnp.float32),
            pltpu.SemaphoreType.DMA,
        ],
        compiler_params=pltpu.CompilerParams(needs_layout_passes=False),
    )()
    return f(wpad, idx_flat, zeros)


def kernel(inputs, weight):
    x = jnp.transpose(inputs, (0, 2, 1)).reshape(_N, _D)

    dist3, enc3, idx, losspart = pl.pallas_call(
        _tc_body,
        grid=(_GRID,),
        in_specs=[
            pl.BlockSpec((_R, _D), lambda i: (i, 0)),
            pl.BlockSpec((_K, _D), lambda i: (0, 0)),
            pl.BlockSpec((1, _K), lambda i: (0, 0)),
            pl.BlockSpec((_R, 1), lambda i: (i, 0)),
        ],
        out_specs=[
            pl.BlockSpec((_SLABS, 1024, 256), lambda i: (i, 0, 0)),
            pl.BlockSpec((_SLABS, 1024, 256), lambda i: (i, 0, 0)),
            pl.BlockSpec((_R, 1), lambda i: (i, 0)),
            pl.BlockSpec((1, 128), lambda i: (0, 0)),
        ],
        out_shape=[
            jax.ShapeDtypeStruct((_D, 1024, 256), jnp.float32),
            jax.ShapeDtypeStruct((_D, 1024, 256), jnp.float32),
            jax.ShapeDtypeStruct((_N, 1), jnp.int32),
            jax.ShapeDtypeStruct((1, 128), jnp.float32),
        ],
    )(x, weight, jnp.sum(weight ** 2, axis=1)[None, :],
      jnp.sum(x ** 2, axis=1)[:, None])

    zeros = jnp.zeros((_K,), jnp.float32)
    wpad = jnp.pad(weight, ((0, 0), (0, 128 - _D)))
    qpad, counts32 = _sc_encode(wpad, idx.reshape(_N), zeros)

    n_elems = jnp.float32(_N * _D)
    e_latent = losspart[0, 0] / n_elems
    vq_loss = e_latent + 0.25 * e_latent

    counts = jnp.sum(counts32, axis=0)
    avg_probs = counts / jnp.float32(_N)
    perplexity = jnp.exp(-jnp.sum(avg_probs * jnp.log(avg_probs + 1e-10)))

    q = qpad[:, :_D]
    quantized_st = jnp.transpose(q.reshape(16, 1024, _D), (0, 2, 1))
    return (vq_loss, quantized_st, perplexity, enc3, dist3, idx)


# speedup vs baseline: 2.1871x; 1.0428x over previous
import functools

import jax
import jax.numpy as jnp
from jax import lax
from jax.experimental import pallas as pl
from jax.experimental.pallas import tpu as pltpu
from jax.experimental.pallas import tpu_sc as plsc

_K = 1024
_D = 64
_N = 16 * 1024
_R = 1024
_GRID = _N // _R
_SLABS = _R * _K // (1024 * 256)

_NC = 2
_NS = 16
_NW = _NC * _NS
_BPW = _N // _NW


def _tc_body(in_ref, w_ref, wsq_ref, xsq_ref, dist_ref, enc_ref, idx_ref,
             loss_ref):
    i = pl.program_id(0)
    xb = jnp.transpose(in_ref[0], (1, 0))
    w = w_ref[...]
    xsq = xsq_ref[...]
    wsq = wsq_ref[...]
    mm = jax.lax.dot_general(
        xb.astype(jnp.bfloat16), w,
        dimension_numbers=(((1,), (1,)), ((), ())),
        preferred_element_type=jnp.float32)
    dist = xsq + wsq - 2.0 * mm
    dist_ref[...] = dist.reshape(_SLABS, 1024, 256)

    mind = jnp.min(dist, axis=1, keepdims=True)
    kiota = jax.lax.broadcasted_iota(jnp.int32, (_R, _K), 1)
    idx = jnp.min(jnp.where(dist == mind, kiota, _K), axis=1)
    idx_ref[...] = idx[:, None]

    one_hot = (kiota == idx[:, None]).astype(jnp.float32)
    enc_ref[...] = one_hot.reshape(_SLABS, 1024, 256)

    @pl.when(i == 0)
    def _init():
        loss_ref[...] = jnp.zeros_like(loss_ref)

    s = jnp.sum(mind)
    lane = jax.lax.broadcasted_iota(jnp.int32, (1, 128), 1)
    loss_ref[...] += jnp.where(lane == 0, s, 0.0)


def _sc_body(wpad_hbm, idx_hbm, zeros_hbm, qpad_hbm, counts_hbm,
             idx_v, rows_v, counts_v, sem):
    wid = lax.axis_index("s") * _NC + lax.axis_index("c")
    base = wid * _BPW
    pltpu.sync_copy(idx_hbm.at[pl.ds(base, _BPW)], idx_v)

    gathers = [
        pltpu.async_copy(wpad_hbm.at[idx_v.at[pl.ds(128 * j, 128)]],
                         rows_v.at[pl.ds(128 * j, 128), :], sem)
        for j in range(4)
    ]

    pltpu.sync_copy(zeros_hbm, counts_v)
    ones = jnp.ones((16,), jnp.float32)
    for g in range(_BPW // 16):
        cols = idx_v[pl.ds(g * 16, 16)]
        plsc.addupdate_scatter(counts_v, [cols], ones)
    pltpu.sync_copy(counts_v, counts_hbm.at[wid])

    for cp in gathers:
        cp.wait()
    pltpu.sync_copy(rows_v, qpad_hbm.at[pl.ds(base, _BPW), :])


def _sc_encode(wpad, idx_flat, zeros):
    mesh = plsc.VectorSubcoreMesh(core_axis_name="c", subcore_axis_name="s",
                                  num_cores=_NC, num_subcores=_NS)
    f = functools.partial(
        pl.kernel, _sc_body, mesh=mesh,
        out_type=[
            jax.ShapeDtypeStruct((_N, 128), jnp.float32),
            jax.ShapeDtypeStruct((_NW, _K), jnp.float32),
        ],
        scratch_types=[
            pltpu.VMEM((_BPW,), jnp.int32),
            pltpu.VMEM((_BPW, 128), jnp.float32),
            pltpu.VMEM((_K,), jnp.float32),
            pltpu.SemaphoreType.DMA,
        ],
        compiler_params=pltpu.CompilerParams(needs_layout_passes=False),
    )()
    return f(wpad, idx_flat, zeros)


def kernel(inputs, weight):
    x = jnp.transpose(inputs, (0, 2, 1)).reshape(_N, _D)

    dist3, enc3, idx, losspart = pl.pallas_call(
        _tc_body,
        grid=(_GRID,),
        in_specs=[
            pl.BlockSpec((1, _D, 1024), lambda i: (i, 0, 0)),
            pl.BlockSpec((_K, _D), lambda i: (0, 0)),
            pl.BlockSpec((1, _K), lambda i: (0, 0)),
            pl.BlockSpec((_R, 1), lambda i: (i, 0)),
        ],
        out_specs=[
            pl.BlockSpec((_SLABS, 1024, 256), lambda i: (i, 0, 0)),
            pl.BlockSpec((_SLABS, 1024, 256), lambda i: (i, 0, 0)),
            pl.BlockSpec((_R, 1), lambda i: (i, 0)),
            pl.BlockSpec((1, 128), lambda i: (0, 0)),
        ],
        out_shape=[
            jax.ShapeDtypeStruct((_D, 1024, 256), jnp.float32),
            jax.ShapeDtypeStruct((_D, 1024, 256), jnp.float32),
            jax.ShapeDtypeStruct((_N, 1), jnp.int32),
            jax.ShapeDtypeStruct((1, 128), jnp.float32),
        ],
    )(inputs, weight, jnp.sum(weight ** 2, axis=1)[None, :],
      jnp.sum(x ** 2, axis=1)[:, None])

    zeros = jnp.zeros((_K,), jnp.float32)
    wpad = jnp.pad(weight, ((0, 0), (0, 128 - _D)))
    qpad, counts32 = _sc_encode(wpad, idx.reshape(_N), zeros)

    n_elems = jnp.float32(_N * _D)
    e_latent = losspart[0, 0] / n_elems
    vq_loss = e_latent + 0.25 * e_latent

    counts = jnp.sum(counts32, axis=0)
    avg_probs = counts / jnp.float32(_N)
    perplexity = jnp.exp(-jnp.sum(avg_probs * jnp.log(avg_probs + 1e-10)))

    q = qpad[:, :_D]
    quantized_st = jnp.transpose(q.reshape(16, 1024, _D), (0, 2, 1))
    return (vq_loss, quantized_st, perplexity, enc3, dist3, idx)
